# baseline (speedup 1.0000x reference)
import functools

import jax
import jax.numpy as jnp
from jax import lax
from jax.experimental import pallas as pl
from jax.experimental.pallas import tpu as pltpu
from jax.experimental.pallas import tpu_sc as plsc

N_CAT = 26
N_NUM = 13
VOCAB = 100000
EMB_DIM = 32
BATCH = 16384
IN_DIM = N_CAT * EMB_DIM

NW = 32
C = 128
ROWS = BATCH * N_CAT
ROWS_PER_W = ROWS // NW
NCHUNK = ROWS_PER_W // C
NBUF = 4
NGROUP = NCHUNK // NBUF

WIN = 128
NWINF = VOCAB // WIN
NWIN = N_CAT * NWINF
RAG = VOCAB - NWINF * WIN
RAG_VBASE = NWINF * WIN


def _sc_transpose_body(tab_hbm, rag_hbm, out_hbm, win_v, outb_v, s0, s1):
    sems = (s0, s1)
    wid = lax.axis_index("s") * 2 + lax.axis_index("c")
    my_nwin = (NWIN - wid + NW - 1) // NW
    rows0 = lax.iota(jnp.int32, 16)
    rows1 = rows0 + 16

    def start_in(k, b):
        f = k // NWINF
        vbase = pl.multiple_of((k % NWINF) * WIN, WIN)
        pltpu.async_copy(tab_hbm.at[f, :, pl.ds(vbase, WIN)],
                         win_v.at[b], sems[b])

    def transpose_cols(b, off):
        def colgrp(g, _):
            base = g * (16 * EMB_DIM)
            vc0 = jnp.zeros((16,), jnp.int32) + g * 16
            for u in range(16):
                vc = vc0 + u
                a = plsc.load_gather(win_v.at[b], [rows0, vc])
                z = plsc.load_gather(win_v.at[b], [rows1, vc])
                outb_v[pl.ds(base + u * EMB_DIM, 16)] = a
                outb_v[pl.ds(base + u * EMB_DIM + 16, 16)] = z
            return 0

        lax.fori_loop(0, WIN // 16, colgrp, 0)
        pltpu.sync_copy(outb_v, out_hbm.at[pl.ds(off, WIN * EMB_DIM)])

    def handle(k, b):
        f = k // NWINF
        vbase = pl.multiple_of((k % NWINF) * WIN, WIN)
        pltpu.make_async_copy(tab_hbm.at[f, :, pl.ds(vbase, WIN)],
                              win_v.at[b], sems[b]).wait()
        transpose_cols(b, (f * VOCAB + vbase) * EMB_DIM)

    for b in range(2):
        @pl.when(b < my_nwin)
        def _():
            start_in(wid + b * NW, b)

    def group(g, carry):
        for b in range(2):
            i = g * 2 + b

            @pl.when(i < my_nwin)
            def _():
                handle(wid + i * NW, b)

                @pl.when(i + 2 < my_nwin)
                def _():
                    start_in(wid + (i + 2) * NW, b)
        return carry

    lax.fori_loop(0, (NWIN // NW + 2) // 2, group, 0)

    @pl.when(wid < N_CAT)
    def _():
        n = RAG * EMB_DIM
        pltpu.async_copy(rag_hbm.at[pl.ds(wid * n, n)],
                         outb_v.at[pl.ds(0, n)], sems[0])
        pltpu.make_async_copy(rag_hbm.at[pl.ds(wid * n, n)],
                              outb_v.at[pl.ds(0, n)], sems[0]).wait()
        off = (wid * VOCAB + RAG_VBASE) * EMB_DIM
        pltpu.sync_copy(outb_v.at[pl.ds(0, n)],
                        out_hbm.at[pl.ds(off, n)])


@jax.jit
def _sc_transpose(tab_t, rag_flat):
    mesh = plsc.VectorSubcoreMesh(core_axis_name="c", subcore_axis_name="s")
    f = functools.partial(
        pl.kernel,
        out_type=jax.ShapeDtypeStruct((N_CAT * VOCAB * EMB_DIM,), jnp.float32),
        mesh=mesh,
        scratch_types=[
            pltpu.VMEM((2, 32, WIN), jnp.float32),
            pltpu.VMEM((WIN * EMB_DIM,), jnp.float32),
            pltpu.SemaphoreType.DMA,
            pltpu.SemaphoreType.DMA,
        ],
        compiler_params=pltpu.CompilerParams(use_tc_tiling_on_sc=True,
                                             needs_layout_passes=False),
    )(_sc_transpose_body)
    return f(tab_t, rag_flat)


def _sc_gather_body(table_hbm, gidx_hbm, out_hbm, idx_v, rows_v,
                    s0, s1, s2, s3):
    sems = (s0, s1, s2, s3)
    wid = lax.axis_index("s") * 2 + lax.axis_index("c")
    base = wid * ROWS_PER_W
    pltpu.sync_copy(gidx_hbm.at[wid], idx_v)

    for b in range(NBUF):
        pltpu.async_copy(table_hbm.at[idx_v.at[b]], rows_v.at[b], sems[b])

    def group(g, carry):
        for b in range(NBUF):
            j = g * NBUF + b
            pltpu.make_async_copy(
                table_hbm.at[idx_v.at[j]], rows_v.at[b], sems[b]).wait()
            pltpu.sync_copy(rows_v.at[b],
                            out_hbm.at[pl.ds(base + j * C, C)])

            @pl.when(j + NBUF < NCHUNK)
            def _():
                pltpu.async_copy(table_hbm.at[idx_v.at[j + NBUF]],
                                 rows_v.at[b], sems[b])
        return carry

    lax.fori_loop(0, NGROUP, group, 0)


@jax.jit
def _sc_gather(table, gidx):
    mesh = plsc.VectorSubcoreMesh(core_axis_name="c", subcore_axis_name="s")
    f = functools.partial(
        pl.kernel,
        out_type=jax.ShapeDtypeStruct((ROWS, EMB_DIM), jnp.float32),
        mesh=mesh,
        scratch_types=[
            pltpu.VMEM((NCHUNK, C), jnp.int32),
            pltpu.VMEM((NBUF, C, EMB_DIM), jnp.float32),
            pltpu.SemaphoreType.DMA,
            pltpu.SemaphoreType.DMA,
            pltpu.SemaphoreType.DMA,
            pltpu.SemaphoreType.DMA,
        ],
        compiler_params=pltpu.CompilerParams(use_tc_tiling_on_sc=False),
    )(_sc_gather_body)
    return f(table, gidx)


BM = 1024


def _mlp_body(emb_ref, num_ref, w0e_ref, w0n_ref, b0_ref, w1_ref, b1_ref,
              w2_ref, b2_ref, w3_ref, b3_ref, out_ref):
    x = jnp.dot(emb_ref[...], w0e_ref[...], preferred_element_type=jnp.float32)
    x = x + jnp.dot(num_ref[...], w0n_ref[...],
                    preferred_element_type=jnp.float32)
    x = jnp.maximum(x + b0_ref[...], 0.0)
    x = jnp.maximum(
        jnp.dot(x, w1_ref[...], preferred_element_type=jnp.float32)
        + b1_ref[...], 0.0)
    x = jnp.maximum(
        jnp.dot(x, w2_ref[...], preferred_element_type=jnp.float32)
        + b2_ref[...], 0.0)
    logits = jnp.maximum(
        jnp.dot(x, w3_ref[...], preferred_element_type=jnp.float32)
        + b3_ref[...], 0.0)
    l0 = logits[:, 0:1]
    loc = logits[:, 1:2]
    l2 = logits[:, 2:3]
    p = 1.0 / (1.0 + jnp.exp(-l0))
    scale = jnp.maximum(l2, 0.0) + jnp.log1p(jnp.exp(-jnp.abs(l2)))
    out_ref[...] = p * jnp.exp(loc + 0.5 * scale * scale)


@jax.jit
def _mlp(emb, num, w0e, w0n, b0, w1, b1, w2, b2, w3, b3):
    full = lambda i: (0, 0)
    return pl.pallas_call(
        _mlp_body,
        grid=(BATCH // BM,),
        in_specs=[
            pl.BlockSpec((BM, IN_DIM), lambda i: (i, 0)),
            pl.BlockSpec((BM, N_NUM), lambda i: (i, 0)),
            pl.BlockSpec(w0e.shape, full),
            pl.BlockSpec(w0n.shape, full),
            pl.BlockSpec(b0.shape, full),
            pl.BlockSpec(w1.shape, full),
            pl.BlockSpec(b1.shape, full),
            pl.BlockSpec(w2.shape, full),
            pl.BlockSpec(b2.shape, full),
            pl.BlockSpec(w3.shape, full),
            pl.BlockSpec(b3.shape, full),
        ],
        out_specs=pl.BlockSpec((BM, 1), lambda i: (i, 0)),
        out_shape=jax.ShapeDtypeStruct((BATCH, 1), jnp.float32),
    )(emb, num, w0e, w0n, b0, w1, b1, w2, b2, w3, b3)


def kernel(data, emb_tables, W0, b0, W1, b1, W2, b2, W3, b3):
    cat = data[:, :N_CAT].astype(jnp.int32)
    gidx = (cat + jnp.arange(N_CAT, dtype=jnp.int32) * VOCAB)
    gidx = gidx.reshape(NW, NCHUNK, C)
    rag_flat = emb_tables[:, RAG_VBASE:, :].reshape(-1)
    flat = _sc_transpose(emb_tables.transpose(0, 2, 1), rag_flat)
    table = flat.reshape(N_CAT * VOCAB, EMB_DIM)
    emb = _sc_gather(table, gidx).reshape(BATCH, IN_DIM)
    num = data[:, N_CAT:]
    return _mlp(emb, num,
                W0[:IN_DIM], W0[IN_DIM:], b0.reshape(1, -1),
                W1, b1.reshape(1, -1), W2, b2.reshape(1, -1),
                W3, b3.reshape(1, -1))


# speedup vs baseline: 1.0148x; 1.0148x over previous
import functools

import jax
import jax.numpy as jnp
from jax import lax
from jax.experimental import pallas as pl
from jax.experimental.pallas import tpu as pltpu
from jax.experimental.pallas import tpu_sc as plsc

N_CAT = 26
N_NUM = 13
VOCAB = 100000
EMB_DIM = 32
BATCH = 16384
IN_DIM = N_CAT * EMB_DIM

NW = 32
C = 128
ROWS = BATCH * N_CAT
ROWS_PER_W = ROWS // NW
NCHUNK = ROWS_PER_W // C
NBUF = 4
NGROUP = NCHUNK // NBUF

WIN = 512
NWINF = VOCAB // WIN
NWIN = N_CAT * NWINF
RAG = VOCAB - NWINF * WIN
RAG_VBASE = NWINF * WIN


def _sc_transpose_body(tab_hbm, rag_hbm, out_hbm, win_v, outb_v,
                       s0, s1, t0, t1):
    sems = (s0, s1)
    osems = (t0, t1)
    wid = lax.axis_index("s") * 2 + lax.axis_index("c")
    my_nwin = (NWIN - wid + NW - 1) // NW
    rows0 = lax.iota(jnp.int32, 16)
    rows1 = rows0 + 16

    def start_in(k, b):
        f = k // NWINF
        vbase = pl.multiple_of((k % NWINF) * WIN, WIN)
        pltpu.async_copy(tab_hbm.at[f, :, pl.ds(vbase, WIN)],
                         win_v.at[b], sems[b])

    def handle(k, b):
        f = k // NWINF
        vbase = pl.multiple_of((k % NWINF) * WIN, WIN)
        pltpu.make_async_copy(tab_hbm.at[f, :, pl.ds(vbase, WIN)],
                              win_v.at[b], sems[b]).wait()

        def colgrp(g, _):
            base = g * (16 * EMB_DIM)
            vc0 = jnp.zeros((16,), jnp.int32) + g * 16
            for u in range(16):
                vc = vc0 + u
                a = plsc.load_gather(win_v.at[b], [rows0, vc])
                z = plsc.load_gather(win_v.at[b], [rows1, vc])
                outb_v[b, pl.ds(base + u * EMB_DIM, 16)] = a
                outb_v[b, pl.ds(base + u * EMB_DIM + 16, 16)] = z
            return 0

        lax.fori_loop(0, WIN // 16, colgrp, 0)
        off = (f * VOCAB + vbase) * EMB_DIM
        pltpu.async_copy(outb_v.at[b], out_hbm.at[pl.ds(off, WIN * EMB_DIM)],
                         osems[b])

    def wait_out(b):
        pltpu.make_async_copy(outb_v.at[b],
                              out_hbm.at[pl.ds(0, WIN * EMB_DIM)],
                              osems[b]).wait()

    for b in range(2):
        @pl.when(b < my_nwin)
        def _():
            start_in(wid + b * NW, b)

    def group(g, carry):
        for b in range(2):
            i = g * 2 + b

            @pl.when(i < my_nwin)
            def _():
                @pl.when(i >= 2)
                def _():
                    wait_out(b)

                handle(wid + i * NW, b)

                @pl.when(i + 2 < my_nwin)
                def _():
                    start_in(wid + (i + 2) * NW, b)
        return carry

    lax.fori_loop(0, (NWIN // NW + 2) // 2, group, 0)
    for b in range(2):
        @pl.when(my_nwin > b)
        def _():
            wait_out(b)

    @pl.when(wid < N_CAT)
    def _():
        n = RAG * EMB_DIM
        pltpu.async_copy(rag_hbm.at[pl.ds(wid * n, n)],
                         outb_v.at[0, pl.ds(0, n)], sems[0])
        pltpu.make_async_copy(rag_hbm.at[pl.ds(wid * n, n)],
                              outb_v.at[0, pl.ds(0, n)], sems[0]).wait()
        off = (wid * VOCAB + RAG_VBASE) * EMB_DIM
        pltpu.sync_copy(outb_v.at[0, pl.ds(0, n)],
                        out_hbm.at[pl.ds(off, n)])


@jax.jit
def _sc_transpose(tab_t, rag_flat):
    mesh = plsc.VectorSubcoreMesh(core_axis_name="c", subcore_axis_name="s")
    f = functools.partial(
        pl.kernel,
        out_type=jax.ShapeDtypeStruct((N_CAT * VOCAB * EMB_DIM,), jnp.float32),
        mesh=mesh,
        scratch_types=[
            pltpu.VMEM((2, 32, WIN), jnp.float32),
            pltpu.VMEM((2, WIN * EMB_DIM), jnp.float32),
            pltpu.SemaphoreType.DMA,
            pltpu.SemaphoreType.DMA,
            pltpu.SemaphoreType.DMA,
            pltpu.SemaphoreType.DMA,
        ],
        compiler_params=pltpu.CompilerParams(use_tc_tiling_on_sc=True,
                                             needs_layout_passes=False),
    )(_sc_transpose_body)
    return f(tab_t, rag_flat)


def _sc_gather_body(table_hbm, gidx_hbm, out_hbm, idx_v, rows_v,
                    s0, s1, s2, s3):
    sems = (s0, s1, s2, s3)
    wid = lax.axis_index("s") * 2 + lax.axis_index("c")
    base = wid * ROWS_PER_W
    pltpu.sync_copy(gidx_hbm.at[wid], idx_v)

    for b in range(NBUF):
        pltpu.async_copy(table_hbm.at[idx_v.at[b]], rows_v.at[b], sems[b])

    def group(g, carry):
        for b in range(NBUF):
            j = g * NBUF + b
            pltpu.make_async_copy(
                table_hbm.at[idx_v.at[j]], rows_v.at[b], sems[b]).wait()
            pltpu.sync_copy(rows_v.at[b],
                            out_hbm.at[pl.ds(base + j * C, C)])

            @pl.when(j + NBUF < NCHUNK)
            def _():
                pltpu.async_copy(table_hbm.at[idx_v.at[j + NBUF]],
                                 rows_v.at[b], sems[b])
        return carry

    lax.fori_loop(0, NGROUP, group, 0)


@jax.jit
def _sc_gather(table, gidx):
    mesh = plsc.VectorSubcoreMesh(core_axis_name="c", subcore_axis_name="s")
    f = functools.partial(
        pl.kernel,
        out_type=jax.ShapeDtypeStruct((ROWS, EMB_DIM), jnp.float32),
        mesh=mesh,
        scratch_types=[
            pltpu.VMEM((NCHUNK, C), jnp.int32),
            pltpu.VMEM((NBUF, C, EMB_DIM), jnp.float32),
            pltpu.SemaphoreType.DMA,
            pltpu.SemaphoreType.DMA,
            pltpu.SemaphoreType.DMA,
            pltpu.SemaphoreType.DMA,
        ],
        compiler_params=pltpu.CompilerParams(use_tc_tiling_on_sc=False),
    )(_sc_gather_body)
    return f(table, gidx)


BM = 1024


def _mlp_body(emb_ref, num_ref, w0e_ref, w0n_ref, b0_ref, w1_ref, b1_ref,
              w2_ref, b2_ref, w3_ref, b3_ref, out_ref):
    x = jnp.dot(emb_ref[...], w0e_ref[...], preferred_element_type=jnp.float32)
    x = x + jnp.dot(num_ref[...], w0n_ref[...],
                    preferred_element_type=jnp.float32)
    x = jnp.maximum(x + b0_ref[...], 0.0)
    x = jnp.maximum(
        jnp.dot(x, w1_ref[...], preferred_element_type=jnp.float32)
        + b1_ref[...], 0.0)
    x = jnp.maximum(
        jnp.dot(x, w2_ref[...], preferred_element_type=jnp.float32)
        + b2_ref[...], 0.0)
    logits = jnp.maximum(
        jnp.dot(x, w3_ref[...], preferred_element_type=jnp.float32)
        + b3_ref[...], 0.0)
    l0 = logits[:, 0:1]
    loc = logits[:, 1:2]
    l2 = logits[:, 2:3]
    p = 1.0 / (1.0 + jnp.exp(-l0))
    scale = jnp.maximum(l2, 0.0) + jnp.log1p(jnp.exp(-jnp.abs(l2)))
    out_ref[...] = p * jnp.exp(loc + 0.5 * scale * scale)


@jax.jit
def _mlp(emb, num, w0e, w0n, b0, w1, b1, w2, b2, w3, b3):
    full = lambda i: (0, 0)
    return pl.pallas_call(
        _mlp_body,
        grid=(BATCH // BM,),
        in_specs=[
            pl.BlockSpec((BM, IN_DIM), lambda i: (i, 0)),
            pl.BlockSpec((BM, N_NUM), lambda i: (i, 0)),
            pl.BlockSpec(w0e.shape, full),
            pl.BlockSpec(w0n.shape, full),
            pl.BlockSpec(b0.shape, full),
            pl.BlockSpec(w1.shape, full),
            pl.BlockSpec(b1.shape, full),
            pl.BlockSpec(w2.shape, full),
            pl.BlockSpec(b2.shape, full),
            pl.BlockSpec(w3.shape, full),
            pl.BlockSpec(b3.shape, full),
        ],
        out_specs=pl.BlockSpec((BM, 1), lambda i: (i, 0)),
        out_shape=jax.ShapeDtypeStruct((BATCH, 1), jnp.float32),
    )(emb, num, w0e, w0n, b0, w1, b1, w2, b2, w3, b3)


def kernel(data, emb_tables, W0, b0, W1, b1, W2, b2, W3, b3):
    cat = data[:, :N_CAT].astype(jnp.int32)
    gidx = (cat + jnp.arange(N_CAT, dtype=jnp.int32) * VOCAB)
    gidx = gidx.reshape(NW, NCHUNK, C)
    rag_flat = emb_tables[:, RAG_VBASE:, :].reshape(-1)
    flat = _sc_transpose(emb_tables.transpose(0, 2, 1), rag_flat)
    table = flat.reshape(N_CAT * VOCAB, EMB_DIM)
    emb = _sc_gather(table, gidx).reshape(BATCH, IN_DIM)
    num = data[:, N_CAT:]
    return _mlp(emb, num,
                W0[:IN_DIM], W0[IN_DIM:], b0.reshape(1, -1),
                W1, b1.reshape(1, -1), W2, b2.reshape(1, -1),
                W3, b3.reshape(1, -1))


# speedup vs baseline: 1.4500x; 1.4289x over previous
import functools

import jax
import jax.numpy as jnp
from jax import lax
from jax.experimental import pallas as pl
from jax.experimental.pallas import tpu as pltpu
from jax.experimental.pallas import tpu_sc as plsc

N_CAT = 26
N_NUM = 13
VOCAB = 100000
EMB_DIM = 32
BATCH = 16384
IN_DIM = N_CAT * EMB_DIM

WIN = 2048
G = WIN // 128
NBLKF = (VOCAB + WIN - 1) // WIN
VPAD = NBLKF * WIN
TROWS = N_CAT * VPAD

NW = 32
C = 128
ROWS = BATCH * N_CAT
ROWS_PER_W = ROWS // NW
NCHUNK = ROWS_PER_W // C
NBUF = 4
NGROUP = NCHUNK // NBUF


def _tct_body(in_ref, out_ref):
    x = in_ref[0]
    eye = jnp.eye(128, dtype=jnp.float32)
    pieces = [
        lax.dot_general(eye, x[:, 128 * g:128 * (g + 1)],
                        (((1,), (1,)), ((), ())),
                        preferred_element_type=jnp.float32)
        for g in range(G)
    ]
    out_ref[0] = jnp.concatenate(pieces, axis=1)


@jax.jit
def _tc_transpose(tab_t):
    return pl.pallas_call(
        _tct_body,
        grid=(N_CAT, NBLKF),
        in_specs=[pl.BlockSpec((1, EMB_DIM, WIN), lambda f, c: (f, 0, c))],
        out_specs=pl.BlockSpec((1, 128, 32 * G),
                               lambda f, c: (f * NBLKF + c, 0, 0)),
        out_shape=jax.ShapeDtypeStruct((N_CAT * NBLKF, 128, 32 * G),
                                       jnp.float32),
    )(tab_t)


def _row_index(f, v):
    return (f * NBLKF + v // WIN) * WIN + G * (v % 128) + (v % WIN) // 128


def _sc_gather_body(table_hbm, gidx_hbm, out_hbm, idx_v, rows_v,
                    s0, s1, s2, s3):
    sems = (s0, s1, s2, s3)
    wid = lax.axis_index("s") * 2 + lax.axis_index("c")
    base = wid * ROWS_PER_W
    pltpu.sync_copy(gidx_hbm.at[wid], idx_v)

    for b in range(NBUF):
        pltpu.async_copy(table_hbm.at[idx_v.at[b]], rows_v.at[b], sems[b])

    def group(g, carry):
        for b in range(NBUF):
            j = g * NBUF + b
            pltpu.make_async_copy(
                table_hbm.at[idx_v.at[j]], rows_v.at[b], sems[b]).wait()
            pltpu.sync_copy(rows_v.at[b],
                            out_hbm.at[pl.ds(base + j * C, C)])

            @pl.when(j + NBUF < NCHUNK)
            def _():
                pltpu.async_copy(table_hbm.at[idx_v.at[j + NBUF]],
                                 rows_v.at[b], sems[b])
        return carry

    lax.fori_loop(0, NGROUP, group, 0)


@jax.jit
def _sc_gather(table, gidx):
    mesh = plsc.VectorSubcoreMesh(core_axis_name="c", subcore_axis_name="s")
    f = functools.partial(
        pl.kernel,
        out_type=jax.ShapeDtypeStruct((ROWS, EMB_DIM), jnp.float32),
        mesh=mesh,
        scratch_types=[
            pltpu.VMEM((NCHUNK, C), jnp.int32),
            pltpu.VMEM((NBUF, C, EMB_DIM), jnp.float32),
            pltpu.SemaphoreType.DMA,
            pltpu.SemaphoreType.DMA,
            pltpu.SemaphoreType.DMA,
            pltpu.SemaphoreType.DMA,
        ],
        compiler_params=pltpu.CompilerParams(use_tc_tiling_on_sc=False),
    )(_sc_gather_body)
    return f(table, gidx)


BM = 1024


def _mlp_body(emb_ref, num_ref, w0e_ref, w0n_ref, b0_ref, w1_ref, b1_ref,
              w2_ref, b2_ref, w3_ref, b3_ref, out_ref):
    x = jnp.dot(emb_ref[...], w0e_ref[...], preferred_element_type=jnp.float32)
    x = x + jnp.dot(num_ref[...], w0n_ref[...],
                    preferred_element_type=jnp.float32)
    x = jnp.maximum(x + b0_ref[...], 0.0)
    x = jnp.maximum(
        jnp.dot(x, w1_ref[...], preferred_element_type=jnp.float32)
        + b1_ref[...], 0.0)
    x = jnp.maximum(
        jnp.dot(x, w2_ref[...], preferred_element_type=jnp.float32)
        + b2_ref[...], 0.0)
    logits = jnp.maximum(
        jnp.dot(x, w3_ref[...], preferred_element_type=jnp.float32)
        + b3_ref[...], 0.0)
    l0 = logits[:, 0:1]
    loc = logits[:, 1:2]
    l2 = logits[:, 2:3]
    p = 1.0 / (1.0 + jnp.exp(-l0))
    scale = jnp.maximum(l2, 0.0) + jnp.log1p(jnp.exp(-jnp.abs(l2)))
    out_ref[...] = p * jnp.exp(loc + 0.5 * scale * scale)


@jax.jit
def _mlp(emb, num, w0e, w0n, b0, w1, b1, w2, b2, w3, b3):
    full = lambda i: (0, 0)
    return pl.pallas_call(
        _mlp_body,
        grid=(BATCH // BM,),
        in_specs=[
            pl.BlockSpec((BM, IN_DIM), lambda i: (i, 0)),
            pl.BlockSpec((BM, N_NUM), lambda i: (i, 0)),
            pl.BlockSpec(w0e.shape, full),
            pl.BlockSpec(w0n.shape, full),
            pl.BlockSpec(b0.shape, full),
            pl.BlockSpec(w1.shape, full),
            pl.BlockSpec(b1.shape, full),
            pl.BlockSpec(w2.shape, full),
            pl.BlockSpec(b2.shape, full),
            pl.BlockSpec(w3.shape, full),
            pl.BlockSpec(b3.shape, full),
        ],
        out_specs=pl.BlockSpec((BM, 1), lambda i: (i, 0)),
        out_shape=jax.ShapeDtypeStruct((BATCH, 1), jnp.float32),
    )(emb, num, w0e, w0n, b0, w1, b1, w2, b2, w3, b3)


def kernel(data, emb_tables, W0, b0, W1, b1, W2, b2, W3, b3):
    cat = data[:, :N_CAT].astype(jnp.int32)
    gidx = _row_index(jnp.arange(N_CAT, dtype=jnp.int32)[None, :], cat)
    gidx = gidx.reshape(NW, NCHUNK, C)
    table = _tc_transpose(emb_tables.transpose(0, 2, 1))
    table = table.reshape(TROWS, EMB_DIM)
    emb = _sc_gather(table, gidx).reshape(BATCH, IN_DIM)
    num = data[:, N_CAT:]
    return _mlp(emb, num,
                W0[:IN_DIM], W0[IN_DIM:], b0.reshape(1, -1),
                W1, b1.reshape(1, -1), W2, b2.reshape(1, -1),
                W3, b3.reshape(1, -1))


# speedup vs baseline: 1.8970x; 1.3082x over previous
import functools

import jax
import jax.numpy as jnp
from jax import lax
from jax.experimental import pallas as pl
from jax.experimental.pallas import tpu as pltpu
from jax.experimental.pallas import tpu_sc as plsc

N_CAT = 26
N_NUM = 13
VOCAB = 100000
EMB_DIM = 32
BATCH = 16384
IN_DIM = N_CAT * EMB_DIM

WIN = 2048
G = WIN // 128
NBLKF = (VOCAB + WIN - 1) // WIN
VPAD = NBLKF * WIN
TROWS = N_CAT * VPAD

NW = 32
C = 128
ROWS = BATCH * N_CAT
ROWS_PER_W = ROWS // NW
NCHUNK = ROWS_PER_W // C
NBUF = 4
NGROUP = NCHUNK // NBUF


def _tct_body(in_ref, out_ref):
    x = in_ref[0]
    eye = jnp.eye(128, dtype=jnp.float32)
    vc = jnp.minimum(WIN, VOCAB - pl.program_id(1) * WIN)
    rowg = lax.broadcasted_iota(jnp.int32, (128, 128), 0) // EMB_DIM
    col = lax.broadcasted_iota(jnp.int32, (128, 128), 1)
    rows = []
    for k in range(G // 4):
        xk = jnp.concatenate(
            [x[:, 128 * (4 * k + p):128 * (4 * k + p + 1)] for p in range(4)],
            axis=0)
        xk = jnp.where(128 * (4 * k + rowg) + col < vc, xk, 0.0)
        rows.append(lax.dot_general(xk, eye, (((0,), (0,)), ((), ())),
                                    preferred_element_type=jnp.float32))
    out_ref[0] = jnp.concatenate(rows, axis=0)


@jax.jit
def _tc_transpose(tab_t):
    return pl.pallas_call(
        _tct_body,
        grid=(N_CAT, NBLKF),
        in_specs=[pl.BlockSpec((1, EMB_DIM, WIN), lambda f, c: (f, 0, c))],
        out_specs=pl.BlockSpec((1, WIN // 4, 128),
                               lambda f, c: (f * NBLKF + c, 0, 0)),
        out_shape=jax.ShapeDtypeStruct((N_CAT * NBLKF, WIN // 4, 128),
                                       jnp.float32),
    )(tab_t)


def _row_index(f, v):
    return ((f * NBLKF + v // WIN) * WIN + 512 * ((v % WIN) // 512)
            + 4 * (v % 128) + ((v % WIN) // 128) % 4)


def _sc_gather_body(table_hbm, gidx_hbm, out_hbm, idx_v, rows_v,
                    s0, s1, s2, s3):
    sems = (s0, s1, s2, s3)
    wid = lax.axis_index("s") * 2 + lax.axis_index("c")
    base = wid * ROWS_PER_W
    pltpu.sync_copy(gidx_hbm.at[wid], idx_v)

    for b in range(NBUF):
        pltpu.async_copy(table_hbm.at[idx_v.at[b]], rows_v.at[b], sems[b])

    def group(g, carry):
        for b in range(NBUF):
            j = g * NBUF + b
            pltpu.make_async_copy(
                table_hbm.at[idx_v.at[j]], rows_v.at[b], sems[b]).wait()
            pltpu.sync_copy(rows_v.at[b],
                            out_hbm.at[pl.ds(base + j * C, C)])

            @pl.when(j + NBUF < NCHUNK)
            def _():
                pltpu.async_copy(table_hbm.at[idx_v.at[j + NBUF]],
                                 rows_v.at[b], sems[b])
        return carry

    lax.fori_loop(0, NGROUP, group, 0)


@jax.jit
def _sc_gather(table, gidx):
    mesh = plsc.VectorSubcoreMesh(core_axis_name="c", subcore_axis_name="s")
    f = functools.partial(
        pl.kernel,
        out_type=jax.ShapeDtypeStruct((ROWS, EMB_DIM), jnp.float32),
        mesh=mesh,
        scratch_types=[
            pltpu.VMEM((NCHUNK, C), jnp.int32),
            pltpu.VMEM((NBUF, C, EMB_DIM), jnp.float32),
            pltpu.SemaphoreType.DMA,
            pltpu.SemaphoreType.DMA,
            pltpu.SemaphoreType.DMA,
            pltpu.SemaphoreType.DMA,
        ],
        compiler_params=pltpu.CompilerParams(use_tc_tiling_on_sc=False),
    )(_sc_gather_body)
    return f(table, gidx)


BM = 1024


def _mlp_body(emb_ref, num_ref, w0e_ref, w0n_ref, b0_ref, w1_ref, b1_ref,
              w2_ref, b2_ref, w3_ref, b3_ref, out_ref):
    x = jnp.dot(emb_ref[...], w0e_ref[...], preferred_element_type=jnp.float32)
    x = x + jnp.dot(num_ref[...], w0n_ref[...],
                    preferred_element_type=jnp.float32)
    x = jnp.maximum(x + b0_ref[...], 0.0)
    x = jnp.maximum(
        jnp.dot(x, w1_ref[...], preferred_element_type=jnp.float32)
        + b1_ref[...], 0.0)
    x = jnp.maximum(
        jnp.dot(x, w2_ref[...], preferred_element_type=jnp.float32)
        + b2_ref[...], 0.0)
    logits = jnp.maximum(
        jnp.dot(x, w3_ref[...], preferred_element_type=jnp.float32)
        + b3_ref[...], 0.0)
    l0 = logits[:, 0:1]
    loc = logits[:, 1:2]
    l2 = logits[:, 2:3]
    p = 1.0 / (1.0 + jnp.exp(-l0))
    scale = jnp.maximum(l2, 0.0) + jnp.log1p(jnp.exp(-jnp.abs(l2)))
    out_ref[...] = p * jnp.exp(loc + 0.5 * scale * scale)


@jax.jit
def _mlp(emb, num, w0e, w0n, b0, w1, b1, w2, b2, w3, b3):
    full = lambda i: (0, 0)
    return pl.pallas_call(
        _mlp_body,
        grid=(BATCH // BM,),
        in_specs=[
            pl.BlockSpec((BM, IN_DIM), lambda i: (i, 0)),
            pl.BlockSpec((BM, N_NUM), lambda i: (i, 0)),
            pl.BlockSpec(w0e.shape, full),
            pl.BlockSpec(w0n.shape, full),
            pl.BlockSpec(b0.shape, full),
            pl.BlockSpec(w1.shape, full),
            pl.BlockSpec(b1.shape, full),
            pl.BlockSpec(w2.shape, full),
            pl.BlockSpec(b2.shape, full),
            pl.BlockSpec(w3.shape, full),
            pl.BlockSpec(b3.shape, full),
        ],
        out_specs=pl.BlockSpec((BM, 1), lambda i: (i, 0)),
        out_shape=jax.ShapeDtypeStruct((BATCH, 1), jnp.float32),
    )(emb, num, w0e, w0n, b0, w1, b1, w2, b2, w3, b3)


def kernel(data, emb_tables, W0, b0, W1, b1, W2, b2, W3, b3):
    cat = data[:, :N_CAT].astype(jnp.int32)
    gidx = _row_index(jnp.arange(N_CAT, dtype=jnp.int32)[None, :], cat)
    gidx = gidx.reshape(NW, NCHUNK, C)
    table = _tc_transpose(emb_tables.transpose(0, 2, 1))
    table = table.reshape(TROWS, EMB_DIM)
    emb = _sc_gather(table, gidx).reshape(BATCH, IN_DIM)
    num = data[:, N_CAT:]
    return _mlp(emb, num,
                W0[:IN_DIM], W0[IN_DIM:], b0.reshape(1, -1),
                W1, b1.reshape(1, -1), W2, b2.reshape(1, -1),
                W3, b3.reshape(1, -1))


# speedup vs baseline: 2.7839x; 1.4676x over previous
import functools

import jax
import jax.numpy as jnp
from jax import lax
from jax.experimental import pallas as pl
from jax.experimental.pallas import tpu as pltpu
from jax.experimental.pallas import tpu_sc as plsc

N_CAT = 26
N_NUM = 13
VOCAB = 100000
EMB_DIM = 32
BATCH = 16384
IN_DIM = N_CAT * EMB_DIM

WIN = 4096
G = WIN // 128
NBLKF = (VOCAB + WIN - 1) // WIN
VPAD = NBLKF * WIN
TROWS = N_CAT * VPAD

NW = 32
C = 128
ROWS = BATCH * N_CAT
ROWS_PER_W = ROWS // NW
NCHUNK = ROWS_PER_W // C
NBUF = 4
NGROUP = NCHUNK // NBUF


def _tct_body(in_ref, out_ref):
    x = in_ref[0]
    eye = jnp.eye(128, dtype=jnp.float32)
    vc = jnp.minimum(WIN, VOCAB - pl.program_id(1) * WIN)
    rowg = lax.broadcasted_iota(jnp.int32, (128, 128), 0) // EMB_DIM
    col = lax.broadcasted_iota(jnp.int32, (128, 128), 1)
    rows = []
    for k in range(G // 4):
        xk = jnp.concatenate(
            [x[:, 128 * (4 * k + p):128 * (4 * k + p + 1)] for p in range(4)],
            axis=0)
        xk = jnp.where(128 * (4 * k + rowg) + col < vc, xk, 0.0)
        rows.append(lax.dot_general(xk, eye, (((0,), (0,)), ((), ())),
                                    preferred_element_type=jnp.float32))
    out_ref[0] = jnp.concatenate(rows, axis=0)


@jax.jit
def _tc_transpose(tab_t):
    return pl.pallas_call(
        _tct_body,
        grid=(N_CAT, NBLKF),
        in_specs=[pl.BlockSpec((1, EMB_DIM, WIN), lambda f, c: (f, 0, c))],
        out_specs=pl.BlockSpec((1, WIN // 4, 128),
                               lambda f, c: (f * NBLKF + c, 0, 0)),
        out_shape=jax.ShapeDtypeStruct((N_CAT * NBLKF, WIN // 4, 128),
                                       jnp.float32),
    )(tab_t)


def _row_index(f, v):
    return ((f * NBLKF + v // WIN) * WIN + 512 * ((v % WIN) // 512)
            + 4 * (v % 128) + ((v % WIN) // 128) % 4)


def _sc_gather_body(table_hbm, gidx_hbm, out_hbm, idx_v, rows_v,
                    s0, s1, s2, s3):
    sems = (s0, s1, s2, s3)
    wid = lax.axis_index("s") * 2 + lax.axis_index("c")
    base = wid * ROWS_PER_W
    pltpu.sync_copy(gidx_hbm.at[wid], idx_v)

    for b in range(NBUF):
        pltpu.async_copy(table_hbm.at[idx_v.at[b]], rows_v.at[b], sems[b])

    def group(g, carry):
        for b in range(NBUF):
            j = g * NBUF + b
            pltpu.make_async_copy(
                table_hbm.at[idx_v.at[j]], rows_v.at[b], sems[b]).wait()
            pltpu.sync_copy(rows_v.at[b],
                            out_hbm.at[pl.ds(base + j * C, C)])

            @pl.when(j + NBUF < NCHUNK)
            def _():
                pltpu.async_copy(table_hbm.at[idx_v.at[j + NBUF]],
                                 rows_v.at[b], sems[b])
        return carry

    lax.fori_loop(0, NGROUP, group, 0)


@jax.jit
def _sc_gather(table, gidx):
    mesh = plsc.VectorSubcoreMesh(core_axis_name="c", subcore_axis_name="s")
    f = functools.partial(
        pl.kernel,
        out_type=jax.ShapeDtypeStruct((ROWS, EMB_DIM), jnp.float32),
        mesh=mesh,
        scratch_types=[
            pltpu.VMEM((NCHUNK, C), jnp.int32),
            pltpu.VMEM((NBUF, C, EMB_DIM), jnp.float32),
            pltpu.SemaphoreType.DMA,
            pltpu.SemaphoreType.DMA,
            pltpu.SemaphoreType.DMA,
            pltpu.SemaphoreType.DMA,
        ],
        compiler_params=pltpu.CompilerParams(use_tc_tiling_on_sc=False),
    )(_sc_gather_body)
    return f(table, gidx)


BM = 1024


def _mlp_body(emb_ref, num_ref, w0e_ref, w0n_ref, b0_ref, w1_ref, b1_ref,
              w2_ref, b2_ref, w3_ref, b3_ref, out_ref):
    x = jnp.dot(emb_ref[...], w0e_ref[...], preferred_element_type=jnp.float32)
    x = x + jnp.dot(num_ref[...], w0n_ref[...],
                    preferred_element_type=jnp.float32)
    x = jnp.maximum(x + b0_ref[...], 0.0)
    x = jnp.maximum(
        jnp.dot(x, w1_ref[...], preferred_element_type=jnp.float32)
        + b1_ref[...], 0.0)
    x = jnp.maximum(
        jnp.dot(x, w2_ref[...], preferred_element_type=jnp.float32)
        + b2_ref[...], 0.0)
    logits = jnp.maximum(
        jnp.dot(x, w3_ref[...], preferred_element_type=jnp.float32)
        + b3_ref[...], 0.0)
    l0 = logits[:, 0:1]
    loc = logits[:, 1:2]
    l2 = logits[:, 2:3]
    p = 1.0 / (1.0 + jnp.exp(-l0))
    scale = jnp.maximum(l2, 0.0) + jnp.log1p(jnp.exp(-jnp.abs(l2)))
    out_ref[...] = p * jnp.exp(loc + 0.5 * scale * scale)


@jax.jit
def _mlp(emb, num, w0e, w0n, b0, w1, b1, w2, b2, w3, b3):
    full = lambda i: (0, 0)
    return pl.pallas_call(
        _mlp_body,
        grid=(BATCH // BM,),
        in_specs=[
            pl.BlockSpec((BM, IN_DIM), lambda i: (i, 0)),
            pl.BlockSpec((BM, N_NUM), lambda i: (i, 0)),
            pl.BlockSpec(w0e.shape, full),
            pl.BlockSpec(w0n.shape, full),
            pl.BlockSpec(b0.shape, full),
            pl.BlockSpec(w1.shape, full),
            pl.BlockSpec(b1.shape, full),
            pl.BlockSpec(w2.shape, full),
            pl.BlockSpec(b2.shape, full),
            pl.BlockSpec(w3.shape, full),
            pl.BlockSpec(b3.shape, full),
        ],
        out_specs=pl.BlockSpec((BM, 1), lambda i: (i, 0)),
        out_shape=jax.ShapeDtypeStruct((BATCH, 1), jnp.float32),
    )(emb, num, w0e, w0n, b0, w1, b1, w2, b2, w3, b3)


def kernel(data, emb_tables, W0, b0, W1, b1, W2, b2, W3, b3):
    cat = data[:, :N_CAT].astype(jnp.int32)
    gidx = _row_index(jnp.arange(N_CAT, dtype=jnp.int32)[None, :], cat)
    gidx = gidx.reshape(NW, NCHUNK, C)
    table = _tc_transpose(emb_tables.transpose(0, 2, 1))
    table = table.reshape(TROWS, EMB_DIM)
    emb = _sc_gather(table, gidx).reshape(BATCH, IN_DIM)
    num = data[:, N_CAT:]
    return _mlp(emb, num,
                W0[:IN_DIM], W0[IN_DIM:], b0.reshape(1, -1),
                W1, b1.reshape(1, -1), W2, b2.reshape(1, -1),
                W3, b3.reshape(1, -1))


# speedup vs baseline: 3.5547x; 1.2769x over previous
import functools

import jax
import jax.numpy as jnp
from jax import lax
from jax.experimental import pallas as pl
from jax.experimental.pallas import tpu as pltpu
from jax.experimental.pallas import tpu_sc as plsc

N_CAT = 26
N_NUM = 13
VOCAB = 100000
EMB_DIM = 32
BATCH = 16384
IN_DIM = N_CAT * EMB_DIM

WIN = 8192
G = WIN // 128
NBLKF = (VOCAB + WIN - 1) // WIN
VPAD = NBLKF * WIN
TROWS = N_CAT * VPAD

NW = 32
C = 128
ROWS = BATCH * N_CAT
ROWS_PER_W = ROWS // NW
NCHUNK = ROWS_PER_W // C
NBUF = 4
NGROUP = NCHUNK // NBUF


def _tct_body(in_ref, out_ref):
    x = in_ref[0]
    eye = jnp.eye(128, dtype=jnp.float32)
    vc = jnp.minimum(WIN, VOCAB - pl.program_id(1) * WIN)
    rowg = lax.broadcasted_iota(jnp.int32, (128, 128), 0) // EMB_DIM
    col = lax.broadcasted_iota(jnp.int32, (128, 128), 1)
    rows = []
    for k in range(G // 4):
        xk = jnp.concatenate(
            [x[:, 128 * (4 * k + p):128 * (4 * k + p + 1)] for p in range(4)],
            axis=0)
        xk = jnp.where(128 * (4 * k + rowg) + col < vc, xk, 0.0)
        rows.append(lax.dot_general(xk, eye, (((0,), (0,)), ((), ())),
                                    preferred_element_type=jnp.float32))
    out_ref[0] = jnp.concatenate(rows, axis=0)


@jax.jit
def _tc_transpose(tab_t):
    return pl.pallas_call(
        _tct_body,
        grid=(N_CAT, NBLKF),
        in_specs=[pl.BlockSpec((1, EMB_DIM, WIN), lambda f, c: (f, 0, c))],
        out_specs=pl.BlockSpec((1, WIN // 4, 128),
                               lambda f, c: (f * NBLKF + c, 0, 0)),
        out_shape=jax.ShapeDtypeStruct((N_CAT * NBLKF, WIN // 4, 128),
                                       jnp.float32),
    )(tab_t)


def _row_index(f, v):
    return ((f * NBLKF + v // WIN) * WIN + 512 * ((v % WIN) // 512)
            + 4 * (v % 128) + ((v % WIN) // 128) % 4)


def _sc_gather_body(table_hbm, gidx_hbm, out_hbm, idx_v, rows_v,
                    s0, s1, s2, s3):
    sems = (s0, s1, s2, s3)
    wid = lax.axis_index("s") * 2 + lax.axis_index("c")
    base = wid * ROWS_PER_W
    pltpu.sync_copy(gidx_hbm.at[wid], idx_v)

    for b in range(NBUF):
        pltpu.async_copy(table_hbm.at[idx_v.at[b]], rows_v.at[b], sems[b])

    def group(g, carry):
        for b in range(NBUF):
            j = g * NBUF + b
            pltpu.make_async_copy(
                table_hbm.at[idx_v.at[j]], rows_v.at[b], sems[b]).wait()
            pltpu.sync_copy(rows_v.at[b],
                            out_hbm.at[pl.ds(base + j * C, C)])

            @pl.when(j + NBUF < NCHUNK)
            def _():
                pltpu.async_copy(table_hbm.at[idx_v.at[j + NBUF]],
                                 rows_v.at[b], sems[b])
        return carry

    lax.fori_loop(0, NGROUP, group, 0)


@jax.jit
def _sc_gather(table, gidx):
    mesh = plsc.VectorSubcoreMesh(core_axis_name="c", subcore_axis_name="s")
    f = functools.partial(
        pl.kernel,
        out_type=jax.ShapeDtypeStruct((ROWS, EMB_DIM), jnp.float32),
        mesh=mesh,
        scratch_types=[
            pltpu.VMEM((NCHUNK, C), jnp.int32),
            pltpu.VMEM((NBUF, C, EMB_DIM), jnp.float32),
            pltpu.SemaphoreType.DMA,
            pltpu.SemaphoreType.DMA,
            pltpu.SemaphoreType.DMA,
            pltpu.SemaphoreType.DMA,
        ],
        compiler_params=pltpu.CompilerParams(use_tc_tiling_on_sc=False),
    )(_sc_gather_body)
    return f(table, gidx)


BM = 1024


def _mlp_body(emb_ref, num_ref, w0e_ref, w0n_ref, b0_ref, w1_ref, b1_ref,
              w2_ref, b2_ref, w3_ref, b3_ref, out_ref):
    x = jnp.dot(emb_ref[...], w0e_ref[...], preferred_element_type=jnp.float32)
    x = x + jnp.dot(num_ref[...], w0n_ref[...],
                    preferred_element_type=jnp.float32)
    x = jnp.maximum(x + b0_ref[...], 0.0)
    x = jnp.maximum(
        jnp.dot(x, w1_ref[...], preferred_element_type=jnp.float32)
        + b1_ref[...], 0.0)
    x = jnp.maximum(
        jnp.dot(x, w2_ref[...], preferred_element_type=jnp.float32)
        + b2_ref[...], 0.0)
    logits = jnp.maximum(
        jnp.dot(x, w3_ref[...], preferred_element_type=jnp.float32)
        + b3_ref[...], 0.0)
    l0 = logits[:, 0:1]
    loc = logits[:, 1:2]
    l2 = logits[:, 2:3]
    p = 1.0 / (1.0 + jnp.exp(-l0))
    scale = jnp.maximum(l2, 0.0) + jnp.log1p(jnp.exp(-jnp.abs(l2)))
    out_ref[...] = p * jnp.exp(loc + 0.5 * scale * scale)


@jax.jit
def _mlp(emb, num, w0e, w0n, b0, w1, b1, w2, b2, w3, b3):
    full = lambda i: (0, 0)
    return pl.pallas_call(
        _mlp_body,
        grid=(BATCH // BM,),
        in_specs=[
            pl.BlockSpec((BM, IN_DIM), lambda i: (i, 0)),
            pl.BlockSpec((BM, N_NUM), lambda i: (i, 0)),
            pl.BlockSpec(w0e.shape, full),
            pl.BlockSpec(w0n.shape, full),
            pl.BlockSpec(b0.shape, full),
            pl.BlockSpec(w1.shape, full),
            pl.BlockSpec(b1.shape, full),
            pl.BlockSpec(w2.shape, full),
            pl.BlockSpec(b2.shape, full),
            pl.BlockSpec(w3.shape, full),
            pl.BlockSpec(b3.shape, full),
        ],
        out_specs=pl.BlockSpec((BM, 1), lambda i: (i, 0)),
        out_shape=jax.ShapeDtypeStruct((BATCH, 1), jnp.float32),
    )(emb, num, w0e, w0n, b0, w1, b1, w2, b2, w3, b3)


def kernel(data, emb_tables, W0, b0, W1, b1, W2, b2, W3, b3):
    cat = data[:, :N_CAT].astype(jnp.int32)
    gidx = _row_index(jnp.arange(N_CAT, dtype=jnp.int32)[None, :], cat)
    gidx = gidx.reshape(NW, NCHUNK, C)
    table = _tc_transpose(emb_tables.transpose(0, 2, 1))
    table = table.reshape(TROWS, EMB_DIM)
    emb = _sc_gather(table, gidx).reshape(BATCH, IN_DIM)
    num = data[:, N_CAT:]
    return _mlp(emb, num,
                W0[:IN_DIM], W0[IN_DIM:], b0.reshape(1, -1),
                W1, b1.reshape(1, -1), W2, b2.reshape(1, -1),
                W3, b3.reshape(1, -1))


# speedup vs baseline: 4.2159x; 1.1860x over previous
import functools

import jax
import jax.numpy as jnp
from jax import lax
from jax.experimental import pallas as pl
from jax.experimental.pallas import tpu as pltpu
from jax.experimental.pallas import tpu_sc as plsc

N_CAT = 26
N_NUM = 13
VOCAB = 100000
EMB_DIM = 32
BATCH = 16384
IN_DIM = N_CAT * EMB_DIM

WIN = 16384
G = WIN // 128
NBLKF = (VOCAB + WIN - 1) // WIN
VPAD = NBLKF * WIN
TROWS = N_CAT * VPAD

NW = 32
C = 128
ROWS = BATCH * N_CAT
ROWS_PER_W = ROWS // NW
NCHUNK = ROWS_PER_W // C
NBUF = 4
NGROUP = NCHUNK // NBUF


def _tct_body(in_ref, out_ref):
    x = in_ref[0]
    eye = jnp.eye(128, dtype=jnp.float32)
    vc = jnp.minimum(WIN, VOCAB - pl.program_id(1) * WIN)
    rowg = lax.broadcasted_iota(jnp.int32, (128, 128), 0) // EMB_DIM
    col = lax.broadcasted_iota(jnp.int32, (128, 128), 1)
    rows = []
    for k in range(G // 4):
        xk = jnp.concatenate(
            [x[:, 128 * (4 * k + p):128 * (4 * k + p + 1)] for p in range(4)],
            axis=0)
        xk = jnp.where(128 * (4 * k + rowg) + col < vc, xk, 0.0)
        rows.append(lax.dot_general(xk, eye, (((0,), (0,)), ((), ())),
                                    preferred_element_type=jnp.float32))
    out_ref[0] = jnp.concatenate(rows, axis=0)


@jax.jit
def _tc_transpose(tab_t):
    return pl.pallas_call(
        _tct_body,
        grid=(N_CAT, NBLKF),
        in_specs=[pl.BlockSpec((1, EMB_DIM, WIN), lambda f, c: (f, 0, c))],
        out_specs=pl.BlockSpec((1, WIN // 4, 128),
                               lambda f, c: (f * NBLKF + c, 0, 0)),
        out_shape=jax.ShapeDtypeStruct((N_CAT * NBLKF, WIN // 4, 128),
                                       jnp.float32),
    )(tab_t)


def _row_index(f, v):
    return ((f * NBLKF + v // WIN) * WIN + 512 * ((v % WIN) // 512)
            + 4 * (v % 128) + ((v % WIN) // 128) % 4)


def _sc_gather_body(table_hbm, gidx_hbm, out_hbm, idx_v, rows_v,
                    s0, s1, s2, s3):
    sems = (s0, s1, s2, s3)
    wid = lax.axis_index("s") * 2 + lax.axis_index("c")
    base = wid * ROWS_PER_W
    pltpu.sync_copy(gidx_hbm.at[wid], idx_v)

    for b in range(NBUF):
        pltpu.async_copy(table_hbm.at[idx_v.at[b]], rows_v.at[b], sems[b])

    def group(g, carry):
        for b in range(NBUF):
            j = g * NBUF + b
            pltpu.make_async_copy(
                table_hbm.at[idx_v.at[j]], rows_v.at[b], sems[b]).wait()
            pltpu.sync_copy(rows_v.at[b],
                            out_hbm.at[pl.ds(base + j * C, C)])

            @pl.when(j + NBUF < NCHUNK)
            def _():
                pltpu.async_copy(table_hbm.at[idx_v.at[j + NBUF]],
                                 rows_v.at[b], sems[b])
        return carry

    lax.fori_loop(0, NGROUP, group, 0)


@jax.jit
def _sc_gather(table, gidx):
    mesh = plsc.VectorSubcoreMesh(core_axis_name="c", subcore_axis_name="s")
    f = functools.partial(
        pl.kernel,
        out_type=jax.ShapeDtypeStruct((ROWS, EMB_DIM), jnp.float32),
        mesh=mesh,
        scratch_types=[
            pltpu.VMEM((NCHUNK, C), jnp.int32),
            pltpu.VMEM((NBUF, C, EMB_DIM), jnp.float32),
            pltpu.SemaphoreType.DMA,
            pltpu.SemaphoreType.DMA,
            pltpu.SemaphoreType.DMA,
            pltpu.SemaphoreType.DMA,
        ],
        compiler_params=pltpu.CompilerParams(use_tc_tiling_on_sc=False),
    )(_sc_gather_body)
    return f(table, gidx)


BM = 1024


def _mlp_body(emb_ref, num_ref, w0e_ref, w0n_ref, b0_ref, w1_ref, b1_ref,
              w2_ref, b2_ref, w3_ref, b3_ref, out_ref):
    x = jnp.dot(emb_ref[...], w0e_ref[...], preferred_element_type=jnp.float32)
    x = x + jnp.dot(num_ref[...], w0n_ref[...],
                    preferred_element_type=jnp.float32)
    x = jnp.maximum(x + b0_ref[...], 0.0)
    x = jnp.maximum(
        jnp.dot(x, w1_ref[...], preferred_element_type=jnp.float32)
        + b1_ref[...], 0.0)
    x = jnp.maximum(
        jnp.dot(x, w2_ref[...], preferred_element_type=jnp.float32)
        + b2_ref[...], 0.0)
    logits = jnp.maximum(
        jnp.dot(x, w3_ref[...], preferred_element_type=jnp.float32)
        + b3_ref[...], 0.0)
    l0 = logits[:, 0:1]
    loc = logits[:, 1:2]
    l2 = logits[:, 2:3]
    p = 1.0 / (1.0 + jnp.exp(-l0))
    scale = jnp.maximum(l2, 0.0) + jnp.log1p(jnp.exp(-jnp.abs(l2)))
    out_ref[...] = p * jnp.exp(loc + 0.5 * scale * scale)


@jax.jit
def _mlp(emb, num, w0e, w0n, b0, w1, b1, w2, b2, w3, b3):
    full = lambda i: (0, 0)
    return pl.pallas_call(
        _mlp_body,
        grid=(BATCH // BM,),
        in_specs=[
            pl.BlockSpec((BM, IN_DIM), lambda i: (i, 0)),
            pl.BlockSpec((BM, N_NUM), lambda i: (i, 0)),
            pl.BlockSpec(w0e.shape, full),
            pl.BlockSpec(w0n.shape, full),
            pl.BlockSpec(b0.shape, full),
            pl.BlockSpec(w1.shape, full),
            pl.BlockSpec(b1.shape, full),
            pl.BlockSpec(w2.shape, full),
            pl.BlockSpec(b2.shape, full),
            pl.BlockSpec(w3.shape, full),
            pl.BlockSpec(b3.shape, full),
        ],
        out_specs=pl.BlockSpec((BM, 1), lambda i: (i, 0)),
        out_shape=jax.ShapeDtypeStruct((BATCH, 1), jnp.float32),
    )(emb, num, w0e, w0n, b0, w1, b1, w2, b2, w3, b3)


def kernel(data, emb_tables, W0, b0, W1, b1, W2, b2, W3, b3):
    cat = data[:, :N_CAT].astype(jnp.int32)
    gidx = _row_index(jnp.arange(N_CAT, dtype=jnp.int32)[None, :], cat)
    gidx = gidx.reshape(NW, NCHUNK, C)
    table = _tc_transpose(emb_tables.transpose(0, 2, 1))
    table = table.reshape(TROWS, EMB_DIM)
    emb = _sc_gather(table, gidx).reshape(BATCH, IN_DIM)
    num = data[:, N_CAT:]
    return _mlp(emb, num,
                W0[:IN_DIM], W0[IN_DIM:], b0.reshape(1, -1),
                W1, b1.reshape(1, -1), W2, b2.reshape(1, -1),
                W3, b3.reshape(1, -1))


# speedup vs baseline: 4.3284x; 1.0267x over previous
import functools

import jax
import jax.numpy as jnp
from jax import lax
from jax.experimental import pallas as pl
from jax.experimental.pallas import tpu as pltpu
from jax.experimental.pallas import tpu_sc as plsc

N_CAT = 26
N_NUM = 13
VOCAB = 100000
EMB_DIM = 32
BATCH = 16384
IN_DIM = N_CAT * EMB_DIM

WIN = 32768
G = WIN // 128
NBLKF = (VOCAB + WIN - 1) // WIN
VPAD = NBLKF * WIN
TROWS = N_CAT * VPAD

NW = 32
C = 128
ROWS = BATCH * N_CAT
ROWS_PER_W = ROWS // NW
NCHUNK = ROWS_PER_W // C
NBUF = 4
NGROUP = NCHUNK // NBUF


def _tct_body(in_ref, out_ref):
    x = in_ref[0]
    eye = jnp.eye(128, dtype=jnp.float32)
    vc = jnp.minimum(WIN, VOCAB - pl.program_id(1) * WIN)
    rowg = lax.broadcasted_iota(jnp.int32, (128, 128), 0) // EMB_DIM
    col = lax.broadcasted_iota(jnp.int32, (128, 128), 1)
    rows = []
    for k in range(G // 4):
        xk = jnp.concatenate(
            [x[:, 128 * (4 * k + p):128 * (4 * k + p + 1)] for p in range(4)],
            axis=0)
        xk = jnp.where(128 * (4 * k + rowg) + col < vc, xk, 0.0)
        rows.append(lax.dot_general(xk, eye, (((0,), (0,)), ((), ())),
                                    preferred_element_type=jnp.float32))
    out_ref[0] = jnp.concatenate(rows, axis=0)


@jax.jit
def _tc_transpose(tab_t):
    return pl.pallas_call(
        _tct_body,
        grid=(N_CAT, NBLKF),
        in_specs=[pl.BlockSpec((1, EMB_DIM, WIN), lambda f, c: (f, 0, c))],
        out_specs=pl.BlockSpec((1, WIN // 4, 128),
                               lambda f, c: (f * NBLKF + c, 0, 0)),
        out_shape=jax.ShapeDtypeStruct((N_CAT * NBLKF, WIN // 4, 128),
                                       jnp.float32),
    )(tab_t)


def _row_index(f, v):
    return ((f * NBLKF + v // WIN) * WIN + 512 * ((v % WIN) // 512)
            + 4 * (v % 128) + ((v % WIN) // 128) % 4)


def _sc_gather_body(table_hbm, gidx_hbm, out_hbm, idx_v, rows_v,
                    s0, s1, s2, s3):
    sems = (s0, s1, s2, s3)
    wid = lax.axis_index("s") * 2 + lax.axis_index("c")
    base = wid * ROWS_PER_W
    pltpu.sync_copy(gidx_hbm.at[wid], idx_v)

    for b in range(NBUF):
        pltpu.async_copy(table_hbm.at[idx_v.at[b]], rows_v.at[b], sems[b])

    def group(g, carry):
        for b in range(NBUF):
            j = g * NBUF + b
            pltpu.make_async_copy(
                table_hbm.at[idx_v.at[j]], rows_v.at[b], sems[b]).wait()
            pltpu.sync_copy(rows_v.at[b],
                            out_hbm.at[pl.ds(base + j * C, C)])

            @pl.when(j + NBUF < NCHUNK)
            def _():
                pltpu.async_copy(table_hbm.at[idx_v.at[j + NBUF]],
                                 rows_v.at[b], sems[b])
        return carry

    lax.fori_loop(0, NGROUP, group, 0)


@jax.jit
def _sc_gather(table, gidx):
    mesh = plsc.VectorSubcoreMesh(core_axis_name="c", subcore_axis_name="s")
    f = functools.partial(
        pl.kernel,
        out_type=jax.ShapeDtypeStruct((ROWS, EMB_DIM), jnp.float32),
        mesh=mesh,
        scratch_types=[
            pltpu.VMEM((NCHUNK, C), jnp.int32),
            pltpu.VMEM((NBUF, C, EMB_DIM), jnp.float32),
            pltpu.SemaphoreType.DMA,
            pltpu.SemaphoreType.DMA,
            pltpu.SemaphoreType.DMA,
            pltpu.SemaphoreType.DMA,
        ],
        compiler_params=pltpu.CompilerParams(use_tc_tiling_on_sc=False),
    )(_sc_gather_body)
    return f(table, gidx)


BM = 1024


def _mlp_body(emb_ref, num_ref, w0e_ref, w0n_ref, b0_ref, w1_ref, b1_ref,
              w2_ref, b2_ref, w3_ref, b3_ref, out_ref):
    x = jnp.dot(emb_ref[...], w0e_ref[...], preferred_element_type=jnp.float32)
    x = x + jnp.dot(num_ref[...], w0n_ref[...],
                    preferred_element_type=jnp.float32)
    x = jnp.maximum(x + b0_ref[...], 0.0)
    x = jnp.maximum(
        jnp.dot(x, w1_ref[...], preferred_element_type=jnp.float32)
        + b1_ref[...], 0.0)
    x = jnp.maximum(
        jnp.dot(x, w2_ref[...], preferred_element_type=jnp.float32)
        + b2_ref[...], 0.0)
    logits = jnp.maximum(
        jnp.dot(x, w3_ref[...], preferred_element_type=jnp.float32)
        + b3_ref[...], 0.0)
    l0 = logits[:, 0:1]
    loc = logits[:, 1:2]
    l2 = logits[:, 2:3]
    p = 1.0 / (1.0 + jnp.exp(-l0))
    scale = jnp.maximum(l2, 0.0) + jnp.log1p(jnp.exp(-jnp.abs(l2)))
    out_ref[...] = p * jnp.exp(loc + 0.5 * scale * scale)


@jax.jit
def _mlp(emb, num, w0e, w0n, b0, w1, b1, w2, b2, w3, b3):
    full = lambda i: (0, 0)
    return pl.pallas_call(
        _mlp_body,
        grid=(BATCH // BM,),
        in_specs=[
            pl.BlockSpec((BM, IN_DIM), lambda i: (i, 0)),
            pl.BlockSpec((BM, N_NUM), lambda i: (i, 0)),
            pl.BlockSpec(w0e.shape, full),
            pl.BlockSpec(w0n.shape, full),
            pl.BlockSpec(b0.shape, full),
            pl.BlockSpec(w1.shape, full),
            pl.BlockSpec(b1.shape, full),
            pl.BlockSpec(w2.shape, full),
            pl.BlockSpec(b2.shape, full),
            pl.BlockSpec(w3.shape, full),
            pl.BlockSpec(b3.shape, full),
        ],
        out_specs=pl.BlockSpec((BM, 1), lambda i: (i, 0)),
        out_shape=jax.ShapeDtypeStruct((BATCH, 1), jnp.float32),
    )(emb, num, w0e, w0n, b0, w1, b1, w2, b2, w3, b3)


def kernel(data, emb_tables, W0, b0, W1, b1, W2, b2, W3, b3):
    cat = data[:, :N_CAT].astype(jnp.int32)
    gidx = _row_index(jnp.arange(N_CAT, dtype=jnp.int32)[None, :], cat)
    gidx = gidx.reshape(NW, NCHUNK, C)
    table = _tc_transpose(emb_tables.transpose(0, 2, 1))
    table = table.reshape(TROWS, EMB_DIM)
    emb = _sc_gather(table, gidx).reshape(BATCH, IN_DIM)
    num = data[:, N_CAT:]
    return _mlp(emb, num,
                W0[:IN_DIM], W0[IN_DIM:], b0.reshape(1, -1),
                W1, b1.reshape(1, -1), W2, b2.reshape(1, -1),
                W3, b3.reshape(1, -1))
